# baseline (speedup 1.0000x reference)
import functools

import jax
import jax.numpy as jnp
from jax import lax
from jax.experimental import pallas as pl
from jax.experimental.pallas import tpu as pltpu
from jax.experimental.pallas import tpu_sc as plsc

MB = 13


def _band_blocks_kernel(emb_ref, out_ref, *, K, D, NT):
    step = pl.program_id(0)
    for mm in range(MB):
        m = step * MB + mm
        kk = jax.lax.broadcasted_iota(jnp.int32, (K, 256), 0)
        x = jax.lax.broadcasted_iota(jnp.int32, (K, 256), 1)
        pos = jnp.clip(x - m, 0, K - 1)
        pos = jnp.where(m == NT - 2, 0, pos)
        pos = jnp.where(m == NT - 1, K - 1, pos)
        oh = (kk == pos).astype(jnp.float32)
        res = jax.lax.dot_general(
            emb_ref[...], oh, (((0,), (0,)), ((), ())),
            preferred_element_type=jnp.float32)
        out_ref[pl.ds(D * mm, D), :] = res


def _build_sc_kernel(sq, sv, D, n_shift):
    info = plsc.get_sparse_core_info()
    NC = info.num_cores
    NW = NC * info.num_subcores
    P = sq // NW
    ntiles = sv // 128
    LO4, BA, HI1, LO1, BB, HI4 = 0, 512, 768, 896, 1024, 1280
    SLANES = 1792

    mesh = plsc.VectorSubcoreMesh(core_axis_name="c", subcore_axis_name="s")

    @functools.partial(
        pl.kernel, mesh=mesh,
        out_type=jax.ShapeDtypeStruct((sq * D, sv), jnp.float32),
        scratch_types=[
            pltpu.VMEM((D, SLANES), jnp.float32),
            pltpu.SemaphoreType.DMA,
            pltpu.SemaphoreType.DMA,
            pltpu.SemaphoreType.DMA,
            pltpu.SemaphoreType.DMA,
        ],
    )
    def k(b_hbm, out_hbm, s_ref, sem_ba, sem_bb, sem_bs, sem_s):
        wid = lax.axis_index("s") * NC + lax.axis_index("c")
        base = wid * P

        def band_src(mm):
            return b_hbm.at[pl.ds(pl.multiple_of(D * mm, 8), D), :]

        lo_src = b_hbm.at[pl.ds(D * n_shift, D), pl.ds(0, 128)]
        hi_src = b_hbm.at[pl.ds(D * (n_shift + 1), D), pl.ds(0, 128)]
        for lane in (LO4, LO4 + 128, LO4 + 256, LO4 + 384, LO1):
            pltpu.sync_copy(lo_src, s_ref.at[:, pl.ds(lane, 128)])
        for lane in (HI1, HI4, HI4 + 128, HI4 + 256, HI4 + 384):
            pltpu.sync_copy(hi_src, s_ref.at[:, pl.ds(lane, 128)])

        pltpu.async_copy(band_src(lax.rem(base + 64, 128)),
                         s_ref.at[:, pl.ds(BA, 256)], sem_ba)

        def dummy_wait(sem, lanes):
            pltpu.make_async_copy(
                out_hbm.at[pl.ds(0, D), pl.ds(0, lanes)],
                s_ref.at[:, pl.ds(0, lanes)], sem).wait()

        def plane_body(p, _):
            i = base + p
            parity = lax.rem(p, 2)
            m_next = lax.rem(i + 1 + 64, 128)
            t0 = lax.div(i + 64, 128) - 1
            bt = jnp.clip(t0, 0, ntiles - 2)
            row0 = pl.multiple_of(D * i, 8)

            @pl.when(p >= 1)
            def _():
                dummy_wait(sem_bs, 256)

            @pl.when(jnp.logical_and(p + 1 < P, parity == 1))
            def _():
                pltpu.async_copy(band_src(m_next),
                                 s_ref.at[:, pl.ds(BA, 256)], sem_ba)

            @pl.when(jnp.logical_and(p + 1 < P, parity == 0))
            def _():
                pltpu.async_copy(band_src(m_next),
                                 s_ref.at[:, pl.ds(BB, 256)], sem_bb)

            @pl.when(parity == 0)
            def _():
                dummy_wait(sem_ba, 256)

            @pl.when(parity == 1)
            def _():
                dummy_wait(sem_bb, 256)

            slot = BA + (BB - BA) * parity
            soff = pl.multiple_of(slot + 128 * (bt - t0), 128)
            bdst = pl.multiple_of(128 * bt, 128)
            pltpu.async_copy(s_ref.at[:, pl.ds(soff, 256)],
                             out_hbm.at[pl.ds(row0, D), pl.ds(bdst, 256)],
                             sem_bs)

            n4 = lax.div(bt, 4)
            rem = bt - 4 * n4
            for qq in range(3):
                @pl.when(qq < n4)
                def _():
                    pltpu.async_copy(
                        s_ref.at[:, pl.ds(LO4, 512)],
                        out_hbm.at[pl.ds(row0, D), pl.ds(512 * qq, 512)],
                        sem_s)

            lo_tail = pl.multiple_of(512 * n4, 128)

            @pl.when(rem >= 2)
            def _():
                pltpu.async_copy(
                    s_ref.at[:, pl.ds(LO4, 256)],
                    out_hbm.at[pl.ds(row0, D), pl.ds(lo_tail, 256)],
                    sem_s)

            lo_tail2 = pl.multiple_of(lo_tail + 256 * lax.div(rem, 2), 128)

            @pl.when(lax.rem(rem, 2) == 1)
            def _():
                pltpu.async_copy(
                    s_ref.at[:, pl.ds(LO4, 128)],
                    out_hbm.at[pl.ds(row0, D), pl.ds(lo_tail2, 128)],
                    sem_s)

            hs = pl.multiple_of(128 * (bt + 2), 128)
            h0 = (ntiles - 2) - bt
            m4 = lax.div(h0, 4)
            hrem = h0 - 4 * m4
            for qq in range(3):
                @pl.when(qq < m4)
                def _():
                    pltpu.async_copy(
                        s_ref.at[:, pl.ds(HI4, 512)],
                        out_hbm.at[pl.ds(row0, D),
                                   pl.ds(pl.multiple_of(hs + 512 * qq, 128),
                                         512)],
                        sem_s)

            hi_tail = pl.multiple_of(hs + 512 * m4, 128)

            @pl.when(hrem >= 2)
            def _():
                pltpu.async_copy(
                    s_ref.at[:, pl.ds(HI4, 256)],
                    out_hbm.at[pl.ds(row0, D), pl.ds(hi_tail, 256)],
                    sem_s)

            hi_tail2 = pl.multiple_of(hi_tail + 256 * lax.div(hrem, 2), 128)

            @pl.when(lax.rem(hrem, 2) == 1)
            def _():
                pltpu.async_copy(
                    s_ref.at[:, pl.ds(HI4, 128)],
                    out_hbm.at[pl.ds(row0, D), pl.ds(hi_tail2, 128)],
                    sem_s)

            return 0

        lax.fori_loop(0, P, plane_body, 0)

        dummy_wait(sem_bs, 256)

        def drain_body(p, _):
            dummy_wait(sem_s, SLANES)
            return 0

        lax.fori_loop(0, P, drain_body, 0)

    return k


def kernel(q, v, embeddings):
    sq, sv = q.shape[1], v.shape[1]
    K, D = embeddings.shape
    n_shift = 128
    nt = n_shift + 2

    b_all = pl.pallas_call(
        functools.partial(_band_blocks_kernel, K=K, D=D, NT=nt),
        grid=(nt // MB,),
        in_specs=[pl.BlockSpec((K, D), lambda s: (0, 0))],
        out_specs=pl.BlockSpec((MB * D, 256), lambda s: (s, 0)),
        out_shape=jax.ShapeDtypeStruct((nt * D, 256), jnp.float32),
    )(embeddings)

    sck = _build_sc_kernel(sq, sv, D, n_shift)
    out2 = sck(b_all)
    return out2.reshape(sq, D, sv).transpose(0, 2, 1)
